# baseline (speedup 1.0000x reference)
import jax
import jax.numpy as jnp
from jax.experimental import pallas as pl
from jax.experimental.pallas import tpu as pltpu

_NB = 128


def _edge_kernel(wr_ref, idr_ref, pcr_ref, pen_ref, wcb_ref, idcb_ref,
                 res_ref, out_ref, neww_ref, topi_ref):
    i = pl.program_id(0)

    wr = wr_ref[...]
    idr = idr_ref[...]
    logits_r = jnp.where(idr == 1, wr, -1e30)
    m = jnp.max(logits_r)
    er = jnp.exp(logits_r - m)
    denom = jnp.sum(er)
    ws_r = er / denom

    @pl.when(i == 0)
    def _init():
        out_ref[...] = jnp.zeros_like(out_ref)

    blk = res_ref[...]
    lg_blk = jnp.where(idcb_ref[...] == 1, wcb_ref[...], -1e30)
    w_blk = jnp.exp(lg_blk - m) / denom
    out_ref[...] += jnp.sum(blk * w_blk[:, :, None], axis=0)

    @pl.when(i == 0)
    def _topk():
        n = wr.shape[1]
        pen = pen_ref[0, 0]
        pcr = pcr_ref[...]
        sc_r = ws_r - pen * pcr
        sc_r = jnp.where(idr == 1, sc_r, -jnp.inf)
        sc_c = jnp.transpose(sc_r)

        ii = jax.lax.broadcasted_iota(jnp.int32, (n, n), 0)
        jj = jax.lax.broadcasted_iota(jnp.int32, (n, n), 1)
        beats = (sc_c > sc_r) | ((sc_c == sc_r) & (ii < jj))
        rank = jnp.sum(beats.astype(jnp.int32), axis=0, keepdims=True)

        rows = neww_ref.shape[0]
        r_iota = jax.lax.broadcasted_iota(jnp.int32, (rows, n), 0)
        onehot = rank == r_iota
        neww_ref[...] = jnp.sum(jnp.where(onehot, sc_r, 0.0),
                                axis=1, keepdims=True)
        col = jax.lax.broadcasted_iota(jnp.int32, (rows, n), 1)
        topi_ref[...] = jnp.sum(jnp.where(onehot, col, 0),
                                axis=1, keepdims=True)


def kernel(W, W_id, results, prog_cost, penalty, topN):
    N, B, D = results.shape
    K = 256
    wr = W.reshape(1, N)
    idr = W_id.reshape(1, N)
    pcr = prog_cost.reshape(1, N)
    pen = penalty.reshape(1, 1)
    wc = W.reshape(N, 1)
    idc = W_id.reshape(N, 1)

    grid = (N // _NB,)
    out, neww, topi = pl.pallas_call(
        _edge_kernel,
        grid=grid,
        in_specs=[
            pl.BlockSpec((1, N), lambda i: (0, 0)),
            pl.BlockSpec((1, N), lambda i: (0, 0)),
            pl.BlockSpec((1, N), lambda i: (0, 0)),
            pl.BlockSpec((1, 1), lambda i: (0, 0)),
            pl.BlockSpec((_NB, 1), lambda i: (i, 0)),
            pl.BlockSpec((_NB, 1), lambda i: (i, 0)),
            pl.BlockSpec((_NB, B, D), lambda i: (i, 0, 0)),
        ],
        out_specs=[
            pl.BlockSpec((B, D), lambda i: (0, 0)),
            pl.BlockSpec((K, 1), lambda i: (0, 0)),
            pl.BlockSpec((K, 1), lambda i: (0, 0)),
        ],
        out_shape=[
            jax.ShapeDtypeStruct((B, D), jnp.float32),
            jax.ShapeDtypeStruct((K, 1), jnp.float32),
            jax.ShapeDtypeStruct((K, 1), jnp.int32),
        ],
        compiler_params=pltpu.CompilerParams(
            dimension_semantics=("arbitrary",),
        ),
    )(wr, idr, pcr, pen, wc, idc, results)
    return out, neww.reshape(K), topi.reshape(K)


# speedup vs baseline: 1.0122x; 1.0122x over previous
import jax
import jax.numpy as jnp
from jax.experimental import pallas as pl
from jax.experimental.pallas import tpu as pltpu

_NB = 256


def _edge_kernel(wr_ref, idr_ref, pcr_ref, pen_ref, wcb_ref, idcb_ref,
                 res_ref, out_ref, neww_ref, topi_ref):
    j = pl.program_id(0)
    i = pl.program_id(1)

    wr = wr_ref[...]
    idr = idr_ref[...]
    logits_r = jnp.where(idr == 1, wr, -1e30)
    m = jnp.max(logits_r)
    er = jnp.exp(logits_r - m)
    denom = jnp.sum(er)
    ws_r = er / denom

    @pl.when(i == 0)
    def _init():
        out_ref[...] = jnp.zeros_like(out_ref)

    blk = res_ref[...]
    lg_blk = jnp.where(idcb_ref[...] == 1, wcb_ref[...], -1e30)
    w_blk = jnp.exp(lg_blk - m) / denom
    out_ref[...] += jnp.sum(blk * w_blk[:, :, None], axis=0)

    @pl.when(i == 0)
    def _topk():
        n = wr.shape[1]
        pen = pen_ref[0, 0]
        pcr = pcr_ref[...]
        sc_r = ws_r - pen * pcr
        sc_r = jnp.where(idr == 1, sc_r, -jnp.inf)
        sc_c = jnp.transpose(sc_r)

        ii = jax.lax.broadcasted_iota(jnp.int32, (n, n), 0)
        jj = jax.lax.broadcasted_iota(jnp.int32, (n, n), 1)
        beats = (sc_c > sc_r) | ((sc_c == sc_r) & (ii < jj))
        rank = jnp.sum(beats.astype(jnp.int32), axis=0, keepdims=True)

        rows = neww_ref.shape[0]
        r_iota = jax.lax.broadcasted_iota(jnp.int32, (rows, n), 0) + j * rows
        onehot = rank == r_iota
        neww_ref[...] = jnp.sum(jnp.where(onehot, sc_r, 0.0),
                                axis=1, keepdims=True)
        col = jax.lax.broadcasted_iota(jnp.int32, (rows, n), 1)
        topi_ref[...] = jnp.sum(jnp.where(onehot, col, 0),
                                axis=1, keepdims=True)


def kernel(W, W_id, results, prog_cost, penalty, topN):
    N, B, D = results.shape
    K = 256
    Dc = D // 2
    wr = W.reshape(1, N)
    idr = W_id.reshape(1, N)
    pcr = prog_cost.reshape(1, N)
    pen = penalty.reshape(1, 1)
    wc = W.reshape(N, 1)
    idc = W_id.reshape(N, 1)

    grid = (2, N // _NB)
    out, neww, topi = pl.pallas_call(
        _edge_kernel,
        grid=grid,
        in_specs=[
            pl.BlockSpec((1, N), lambda j, i: (0, 0)),
            pl.BlockSpec((1, N), lambda j, i: (0, 0)),
            pl.BlockSpec((1, N), lambda j, i: (0, 0)),
            pl.BlockSpec((1, 1), lambda j, i: (0, 0)),
            pl.BlockSpec((_NB, 1), lambda j, i: (i, 0)),
            pl.BlockSpec((_NB, 1), lambda j, i: (i, 0)),
            pl.BlockSpec((_NB, B, Dc), lambda j, i: (i, 0, j)),
        ],
        out_specs=[
            pl.BlockSpec((B, Dc), lambda j, i: (0, j)),
            pl.BlockSpec((K // 2, 1), lambda j, i: (j, 0)),
            pl.BlockSpec((K // 2, 1), lambda j, i: (j, 0)),
        ],
        out_shape=[
            jax.ShapeDtypeStruct((B, D), jnp.float32),
            jax.ShapeDtypeStruct((K, 1), jnp.float32),
            jax.ShapeDtypeStruct((K, 1), jnp.int32),
        ],
        compiler_params=pltpu.CompilerParams(
            dimension_semantics=("parallel", "arbitrary"),
        ),
    )(wr, idr, pcr, pen, wc, idc, results)
    return out, neww.reshape(K), topi.reshape(K)
